# Initial kernel scaffold; baseline (speedup 1.0000x reference)
import jax
import jax.numpy as jnp
from jax.experimental import pallas as pl


def kernel(x, edge_index, e, W, b):
    raise NotImplementedError("write your pallas kernel here")


# speedup vs baseline: 3.6296x; 3.6296x over previous
import functools

import jax
import jax.numpy as jnp
from jax import lax
from jax.experimental import pallas as pl
from jax.experimental.pallas import tpu as pltpu
from jax.experimental.pallas import tpu_sc as plsc

NC = 2
NS = 16
L = 16
K = 128
NW = NC * NS


def _make_sc_aggregate(n_pad, d, cpw):
    rows_per_tile = n_pad // NS

    mesh = plsc.VectorSubcoreMesh(
        core_axis_name="c", subcore_axis_name="s",
        num_cores=NC, num_subcores=NS)

    @functools.partial(
        pl.kernel,
        out_type=jax.ShapeDtypeStruct((NC, n_pad, d), jnp.float32),
        mesh=mesh,
        scratch_types=[
            pltpu.VMEM((K,), jnp.int32),
            pltpu.VMEM((K,), jnp.int32),
            pltpu.VMEM((K,), jnp.float32),
            pltpu.VMEM((K, d), jnp.float32),
            pltpu.VMEM_SHARED((n_pad, d), jnp.float32),
            pltpu.SemaphoreType.DMA,
        ],
    )
    def sc_aggregate(src_hbm, dst_hbm, e_hbm, x_hbm, out_hbm,
                     src_v, dst_v, e_v, rows_v, acc, sem):
        cid = lax.axis_index("c")
        sid = lax.axis_index("s")
        wid = sid * NC + cid

        zeros16 = jnp.zeros((L,), jnp.float32)

        def zrow(r, carry):
            for k2 in range(d // L):
                rows_v[r, pl.ds(k2 * L, L)] = zeros16
            return carry
        lax.fori_loop(0, K, zrow, 0)
        tile_base = sid * rows_per_tile
        off = 0
        while off < rows_per_tile:
            sz = min(K, rows_per_tile - off)
            pltpu.sync_copy(rows_v.at[pl.ds(0, sz)],
                            acc.at[pl.ds(tile_base + off, sz)])
            off += sz
        plsc.subcore_barrier()

        def chunk_body(cix, carry):
            base = (wid * cpw + cix) * K
            pltpu.sync_copy(src_hbm.at[pl.ds(base, K)], src_v)
            pltpu.sync_copy(dst_hbm.at[pl.ds(base, K)], dst_v)
            pltpu.sync_copy(e_hbm.at[pl.ds(base, K)], e_v)
            pltpu.async_copy(x_hbm.at[src_v], rows_v, sem).wait()

            def scale_grp(g, c2):
                e_vec = e_v[pl.ds(g * L, L)]
                for i in range(L):
                    ev = e_vec[i]
                    r = g * L + i
                    for k2 in range(d // L):
                        sl = pl.ds(k2 * L, L)
                        rows_v[r, sl] = rows_v[r, sl] * ev
                return c2
            lax.fori_loop(0, K // L, scale_grp, 0)

            pltpu.sync_copy(rows_v, acc.at[dst_v], add=True)
            return carry
        lax.fori_loop(0, cpw, chunk_body, 0)

        plsc.subcore_barrier()
        pltpu.sync_copy(acc.at[pl.ds(tile_base, rows_per_tile)],
                        out_hbm.at[cid, pl.ds(tile_base, rows_per_tile)])

    return sc_aggregate


def _combine_body(p_ref, w_ref, b_ref, o_ref):
    s = p_ref[0] + p_ref[1]
    o_ref[...] = lax.dot_general(
        s, w_ref[...], (((1,), (1,)), ((), ())),
        preferred_element_type=jnp.float32) + b_ref[...]


def kernel(x, edge_index, e, W, b):
    n_nodes, d = x.shape
    e_total = edge_index.shape[1]
    src = edge_index[0].astype(jnp.int32)
    dst = edge_index[1].astype(jnp.int32)
    ef = e[:, 0].astype(jnp.float32)

    cpw = -(-e_total // (NW * K))
    e_pad = NW * cpw * K
    pad = e_pad - e_total
    if pad:
        src = jnp.pad(src, (0, pad))
        dst = jnp.pad(dst, (0, pad))
        ef = jnp.pad(ef, (0, pad))

    n_pad = -(-n_nodes // (8 * NS)) * (8 * NS)
    partials = _make_sc_aggregate(n_pad, d, cpw)(src, dst, ef, x)

    blk = 1000
    grid = n_nodes // blk
    out = pl.pallas_call(
        _combine_body,
        grid=(grid,),
        in_specs=[
            pl.BlockSpec((NC, blk, d), lambda i: (0, i, 0)),
            pl.BlockSpec((d, d), lambda i: (0, 0)),
            pl.BlockSpec((1, d), lambda i: (0, 0)),
        ],
        out_specs=pl.BlockSpec((blk, d), lambda i: (i, 0)),
        out_shape=jax.ShapeDtypeStruct((n_nodes, d), jnp.float32),
    )(partials, W, b.reshape(1, d))
    return out
